# baseline (speedup 1.0000x reference)
import functools

import jax
import jax.numpy as jnp
from jax import lax
from jax.experimental import pallas as pl
from jax.experimental.pallas import tpu as pltpu
from jax.experimental.pallas import tpu_sc as plsc

_N = 10000
_E = 320000
_D = 128
_NS = 16
_C = 40
_G = 25
_NG = 10
_EPC = _E // 2
_EPT = _EPC // _NS
_NP = 10112
_RPT = _NP // _NS
_ZR = 8


def _node_kernel(x_ref, w_ref, b_ref, ax_ref, td_ref, ts_ref):
    r = jnp.dot(x_ref[...], w_ref[...], preferred_element_type=jnp.float32)
    r = r + b_ref[...]
    ax_ref[...] = r[:, 0:128]
    td_ref[...] = r[:, 256:384]
    ts_ref[...] = jnp.concatenate([r[:, 384:512], r[:, 128:256]], axis=1)


def _edge_kernel(ea_ref, w_ref, b_ref, ce_ref):
    r = jnp.dot(ea_ref[...], w_ref[...], preferred_element_type=jnp.float32)
    ce_ref[...] = r + b_ref[...]


def _sc_main(dsti, srci, tabS, tabD, ce_hbm,
             e_hbm, num_hbm, den_hbm, est_hbm,
             idxd, idxs, rows_s, rows_d, ce_buf, m_buf,
             zbuf, stats_loc, stats_all,
             acc, stats_stage,
             semS, semD):
    c = lax.axis_index("c")
    s = lax.axis_index("s")
    w = c * _NS + s

    z16 = jnp.zeros((16,), jnp.float32)

    def zrow(r, _):
        for j in range(_D // 16):
            zbuf[r, pl.ds(j * 16, 16)] = z16
        return 0
    lax.fori_loop(0, _ZR, zrow, 0)
    for r in range(8):
        for j in range(_D // 16):
            stats_loc[r, pl.ds(j * 16, 16)] = z16
    base_rows = s * _RPT
    for i in range(_RPT // _ZR):
        pltpu.sync_copy(zbuf, acc.at[pl.ds(base_rows + i * _ZR, _ZR)])
    plsc.subcore_barrier()

    ebase = c * _EPC + s * _EPT

    def group1(g, _):
        pltpu.sync_copy(dsti.at[w * _NG + g], idxd)
        pltpu.sync_copy(srci.at[w * _NG + g], idxs)
        gbase = ebase + g * _G * _C

        def chunk1(k, _):
            base = gbase + k * _C
            cpS = pltpu.async_copy(tabS.at[idxs.at[k]], rows_s, semS)
            cpD = pltpu.async_copy(tabD.at[idxd.at[k]], rows_d, semD)
            pltpu.sync_copy(ce_hbm.at[pl.ds(base, _C)], ce_buf)
            cpS.wait()
            cpD.wait()

            def row(r, _):
                for j in range(_D // 16):
                    sl = pl.ds(j * 16, 16)
                    e = rows_d[r, sl] + rows_s[r, sl] + ce_buf[r, sl]
                    ce_buf[r, sl] = e
                    sg = 1.0 / (1.0 + jnp.exp(-e))
                    m_buf[r, sl] = sg * rows_s[r, pl.ds(_D + j * 16, 16)]
                return 0
            lax.fori_loop(0, _C, row, 0)

            pltpu.sync_copy(ce_buf, e_hbm.at[pl.ds(base, _C)])
            pltpu.sync_copy(m_buf, acc.at[idxd.at[k]], add=True)
            return 0
        lax.fori_loop(0, _G, chunk1, 0)
        return 0
    lax.fori_loop(0, _NG, group1, 0)

    plsc.subcore_barrier()

    nb = c * _NP + base_rows
    pltpu.sync_copy(acc.at[pl.ds(base_rows, _RPT)],
                    num_hbm.at[pl.ds(nb, _RPT)])
    for i in range(_RPT // _ZR):
        pltpu.sync_copy(zbuf, acc.at[pl.ds(base_rows + i * _ZR, _ZR)])
    plsc.subcore_barrier()

    def group2(g, _):
        pltpu.sync_copy(dsti.at[w * _NG + g], idxd)
        gbase = ebase + g * _G * _C

        def chunk2(k, _):
            base = gbase + k * _C
            pltpu.sync_copy(e_hbm.at[pl.ds(base, _C)], ce_buf)

            def row(r, _):
                for j in range(_D // 16):
                    sl = pl.ds(j * 16, 16)
                    e = ce_buf[r, sl]
                    m_buf[r, sl] = 1.0 / (1.0 + jnp.exp(-e))
                    stats_loc[0, sl] = stats_loc[0, sl] + e
                    stats_loc[1, sl] = stats_loc[1, sl] + e * e
                return 0
            lax.fori_loop(0, _C, row, 0)

            pltpu.sync_copy(m_buf, acc.at[idxd.at[k]], add=True)
            return 0
        lax.fori_loop(0, _G, chunk2, 0)
        return 0
    lax.fori_loop(0, _NG, group2, 0)

    plsc.subcore_barrier()
    pltpu.sync_copy(acc.at[pl.ds(base_rows, _RPT)],
                    den_hbm.at[pl.ds(nb, _RPT)])

    pltpu.sync_copy(stats_loc.at[pl.ds(0, 2)], stats_stage.at[s])
    plsc.subcore_barrier()

    @pl.when(s == 0)
    def _():
        pltpu.sync_copy(stats_stage, stats_all)
        for j in range(_D // 16):
            sl = pl.ds(j * 16, 16)
            stats_loc[0, sl] = z16
            stats_loc[1, sl] = z16

        def red(t, _):
            for j in range(_D // 16):
                sl = pl.ds(j * 16, 16)
                stats_loc[0, sl] = stats_loc[0, sl] + stats_all[t, 0, sl]
                stats_loc[1, sl] = stats_loc[1, sl] + stats_all[t, 1, sl]
            return 0
        lax.fori_loop(0, _NS, red, 0)
        pltpu.sync_copy(stats_loc, est_hbm.at[pl.ds(c * 8, 8)])


def _x1_kernel(ax_ref, num_ref, den_ref, xo_ref, st_ref):
    num = num_ref[0] + num_ref[1]
    den = den_ref[0] + den_ref[1]
    xo = ax_ref[...] + num / (den + 1e-6)
    xo_ref[...] = xo
    s1 = jnp.sum(xo, axis=0, keepdims=True)
    s2 = jnp.sum(xo * xo, axis=0, keepdims=True)
    st_ref[0] = jnp.concatenate(
        [s1, s2, jnp.zeros((6, _D), jnp.float32)], axis=0)


def _x2_kernel(xo_ref, st_ref, gx_ref, bx_ref, out_ref):
    st = st_ref[...]
    m = jnp.sum(st[:, 0, :], axis=0, keepdims=True) * (1.0 / _N)
    ms = jnp.sum(st[:, 1, :], axis=0, keepdims=True) * (1.0 / _N)
    var = ms - m * m
    sc = gx_ref[...] * lax.rsqrt(var + 1e-5)
    sh = bx_ref[...] - m * sc
    out_ref[...] = jnp.maximum(xo_ref[...] * sc + sh, 0.0)


def _enorm_kernel(est_ref, ge_ref, be_ref, e_ref, out_ref):
    est = est_ref[...]
    inv_e = 1.0 / _E
    m = (est[0:1] + est[8:9]) * inv_e
    ms = (est[1:2] + est[9:10]) * inv_e
    var = ms - m * m
    sc = ge_ref[...] * lax.rsqrt(var + 1e-5)
    sh = be_ref[...] - m * sc
    out_ref[...] = jnp.maximum(e_ref[...] * sc + sh, 0.0)


def kernel(x, edge_attr, edge_index, WA, bA, WB, bB, WC, bC, WD, bD, WE, bE,
           gamma_x, beta_x, gamma_e, beta_e):
    f32 = jnp.float32

    Wn = jnp.concatenate([WA, WB, WD, WE], axis=1)
    bn = jnp.concatenate([bA, bB, bD, bE]).reshape(1, 512)
    nb = 10
    nr = _N // nb
    ax, td, ts = pl.pallas_call(
        _node_kernel,
        grid=(nb,),
        in_specs=[
            pl.BlockSpec((nr, _D), lambda i: (i, 0)),
            pl.BlockSpec((_D, 512), lambda i: (0, 0)),
            pl.BlockSpec((1, 512), lambda i: (0, 0)),
        ],
        out_specs=[
            pl.BlockSpec((nr, _D), lambda i: (i, 0)),
            pl.BlockSpec((nr, _D), lambda i: (i, 0)),
            pl.BlockSpec((nr, 2 * _D), lambda i: (i, 0)),
        ],
        out_shape=[
            jax.ShapeDtypeStruct((_N, _D), f32),
            jax.ShapeDtypeStruct((_N, _D), f32),
            jax.ShapeDtypeStruct((_N, 2 * _D), f32),
        ],
    )(x, Wn, bn)

    eb = 320
    er = _E // eb
    ce = pl.pallas_call(
        _edge_kernel,
        grid=(eb,),
        in_specs=[
            pl.BlockSpec((er, _D), lambda i: (i, 0)),
            pl.BlockSpec((_D, _D), lambda i: (0, 0)),
            pl.BlockSpec((1, _D), lambda i: (0, 0)),
        ],
        out_specs=pl.BlockSpec((er, _D), lambda i: (i, 0)),
        out_shape=jax.ShapeDtypeStruct((_E, _D), f32),
    )(edge_attr, WC, bC.reshape(1, _D))

    dst = edge_index[1].reshape(2 * _NS * _NG, _G, _C)
    src = edge_index[0].reshape(2 * _NS * _NG, _G, _C)
    mesh = plsc.VectorSubcoreMesh(core_axis_name="c", subcore_axis_name="s")
    sc_call = functools.partial(
        pl.kernel,
        mesh=mesh,
        out_type=[
            jax.ShapeDtypeStruct((_E, _D), f32),
            jax.ShapeDtypeStruct((2 * _NP, _D), f32),
            jax.ShapeDtypeStruct((2 * _NP, _D), f32),
            jax.ShapeDtypeStruct((16, _D), f32),
        ],
        scratch_types=[
            pltpu.VMEM((_G, _C), jnp.int32),
            pltpu.VMEM((_G, _C), jnp.int32),
            pltpu.VMEM((_C, 2 * _D), f32),
            pltpu.VMEM((_C, _D), f32),
            pltpu.VMEM((_C, _D), f32),
            pltpu.VMEM((_C, _D), f32),
            pltpu.VMEM((_ZR, _D), f32),
            pltpu.VMEM((8, _D), f32),
            pltpu.VMEM((_NS, 2, _D), f32),
            pltpu.VMEM_SHARED((_NP, _D), f32),
            pltpu.VMEM_SHARED((_NS, 2, _D), f32),
            pltpu.SemaphoreType.DMA,
            pltpu.SemaphoreType.DMA,
        ],
    )(_sc_main)
    e2, numf, denf, est = sc_call(dst, src, ts, td, ce)

    xo, xst = pl.pallas_call(
        _x1_kernel,
        grid=(nb,),
        in_specs=[
            pl.BlockSpec((nr, _D), lambda i: (i, 0)),
            pl.BlockSpec((2, nr, _D), lambda i: (0, i, 0)),
            pl.BlockSpec((2, nr, _D), lambda i: (0, i, 0)),
        ],
        out_specs=[
            pl.BlockSpec((nr, _D), lambda i: (i, 0)),
            pl.BlockSpec((1, 8, _D), lambda i: (i, 0, 0)),
        ],
        out_shape=[
            jax.ShapeDtypeStruct((_N, _D), f32),
            jax.ShapeDtypeStruct((nb, 8, _D), f32),
        ],
    )(ax, numf.reshape(2, _NP, _D)[:, :_N], denf.reshape(2, _NP, _D)[:, :_N])

    x_out = pl.pallas_call(
        _x2_kernel,
        grid=(nb,),
        in_specs=[
            pl.BlockSpec((nr, _D), lambda i: (i, 0)),
            pl.BlockSpec((nb, 8, _D), lambda i: (0, 0, 0)),
            pl.BlockSpec((1, _D), lambda i: (0, 0)),
            pl.BlockSpec((1, _D), lambda i: (0, 0)),
        ],
        out_specs=pl.BlockSpec((nr, _D), lambda i: (i, 0)),
        out_shape=jax.ShapeDtypeStruct((_N, _D), f32),
    )(xo, xst, gamma_x.reshape(1, _D), beta_x.reshape(1, _D))

    ob = 320
    orr = _E // ob
    e_out = pl.pallas_call(
        _enorm_kernel,
        grid=(ob,),
        in_specs=[
            pl.BlockSpec((16, _D), lambda i: (0, 0)),
            pl.BlockSpec((1, _D), lambda i: (0, 0)),
            pl.BlockSpec((1, _D), lambda i: (0, 0)),
            pl.BlockSpec((orr, _D), lambda i: (i, 0)),
        ],
        out_specs=pl.BlockSpec((orr, _D), lambda i: (i, 0)),
        out_shape=jax.ShapeDtypeStruct((_E, _D), f32),
    )(est, gamma_e.reshape(1, _D), beta_e.reshape(1, _D), e2)

    return (x_out, e_out)
